# baseline (speedup 1.0000x reference)
import jax
import jax.numpy as jnp
from jax.experimental import pallas as pl
from jax.experimental.pallas import tpu as pltpu

_EPS = 1e-05
_FACT = 2.0
_TN1 = 2048
_NC = 16


def _body(pc1_ref, pc2_ref, d1_ref, n1_ref, d2_ref, n2_ref, out_ref,
          d2n_ref, e1_ref, e2_ref, pc2e_ref):
    @pl.when(pl.program_id(0) == 0)
    def _():
        d2n_ref[...] = d2_ref[...] / n2_ref[...]
        n = pc2_ref.shape[1]
        one = jnp.ones((1, n), jnp.float32)
        x1, y1, z1 = pc1_ref[0:1, :], pc1_ref[1:2, :], pc1_ref[2:3, :]
        x2, y2, z2 = pc2_ref[0:1, :], pc2_ref[1:2, :], pc2_ref[2:3, :]
        e2_ref[...] = jnp.concatenate(
            [x2 * x2, -2.0 * x2, one, y2 * y2, -2.0 * y2, one,
             z2 * z2, -2.0 * z2, one], axis=0)
        e1t = jnp.concatenate(
            [one, x1, x1 * x1, one, y1, y1 * y1, one, z1, z1 * z1], axis=0)
        e1_ref[...] = e1t.T
        pc2e_ref[...] = jnp.concatenate([pc2_ref[...], one], axis=0).T

    h = _TN1 // _NC
    i = pl.program_id(0)

    def _mm(k):
        sl = pl.ds(k * h, h)
        gsl = pl.ds(pl.multiple_of(i * _TN1 + k * h, h), h)
        d1n = d1_ref[:, sl] / n1_ref[:, sl]
        ip = jax.lax.dot_general(d1n, d2n_ref[...], (((0,), (0,)), ((), ())),
                                 preferred_element_type=jnp.float32)
        sqd = jnp.dot(e1_ref[gsl, :], e2_ref[...],
                      preferred_element_type=jnp.float32)
        return ip, sqd

    def _soft(ip, sqd):
        dist = jnp.reciprocal(jnp.maximum(sqd, _EPS)) * (ip * ip)
        m = jnp.max(dist, axis=1, keepdims=True)
        p = jnp.exp2((dist - m) * jnp.float32(_FACT * 1.4426950408889634))
        w = p / jnp.sum(p, axis=1, keepdims=True)
        return jnp.dot(w, pc2e_ref[...], preferred_element_type=jnp.float32)

    pend = {0: _mm(0), 1: _mm(1)}
    outs = []
    for k in range(_NC):
        if k + 2 < _NC:
            pend[k + 2] = _mm(k + 2)
        outs.append(_soft(*pend.pop(k)))
    out_ref[...] = jnp.concatenate(outs, axis=0)


def kernel(pc1, pc2, d1, d2):
    n = pc1.shape[1]
    f32 = jnp.float32
    n1 = jnp.maximum(jnp.linalg.norm(d1, axis=0, keepdims=True), 1e-12)
    n2 = jnp.maximum(jnp.linalg.norm(d2, axis=0, keepdims=True), 1e-12)
    s = d2.shape[0]

    out = pl.pallas_call(
        _body,
        grid=(n // _TN1,),
        in_specs=[
            pl.BlockSpec((3, n), lambda i: (0, 0)),
            pl.BlockSpec((3, n), lambda i: (0, 0)),
            pl.BlockSpec((s, _TN1), lambda i: (0, i)),
            pl.BlockSpec((1, _TN1), lambda i: (0, i)),
            pl.BlockSpec((s, n), lambda i: (0, 0)),
            pl.BlockSpec((1, n), lambda i: (0, 0)),
        ],
        out_specs=pl.BlockSpec((_TN1, 4), lambda i: (i, 0)),
        out_shape=jax.ShapeDtypeStruct((n, 4), f32),
        scratch_shapes=[
            pltpu.VMEM((s, n), f32),
            pltpu.VMEM((n, 9), f32),
            pltpu.VMEM((9, n), f32),
            pltpu.VMEM((n, 4), f32),
        ],
    )(pc1, pc2, d1, n1, d2, n2)

    pc_nearest = out[:, :3].T
    indexor = jnp.ones((n,), pc1.dtype)
    return (pc_nearest, indexor)


# speedup vs baseline: 1.3597x; 1.3597x over previous
import jax
import jax.numpy as jnp
from jax.experimental import pallas as pl
from jax.experimental.pallas import tpu as pltpu

_EPS = 1e-05
_FACT = 2.0
_TN1 = 1024
_NC = 4


def _body(pc1_ref, pc2_ref, d1_ref, n1_ref, d2_ref, n2_ref, out_ref,
          d2n_ref, e1_ref, e2_ref, pc2e_ref):
    @pl.when(pl.program_id(0) == 0)
    def _():
        d2n_ref[...] = d2_ref[...] / n2_ref[...]
        n = pc2_ref.shape[1]
        one = jnp.ones((1, n), jnp.float32)
        x1, y1, z1 = pc1_ref[0:1, :], pc1_ref[1:2, :], pc1_ref[2:3, :]
        x2, y2, z2 = pc2_ref[0:1, :], pc2_ref[1:2, :], pc2_ref[2:3, :]
        e2_ref[...] = jnp.concatenate(
            [x2 * x2, -2.0 * x2, one, y2 * y2, -2.0 * y2, one,
             z2 * z2, -2.0 * z2, one], axis=0)
        e1t = jnp.concatenate(
            [one, x1, x1 * x1, one, y1, y1 * y1, one, z1, z1 * z1], axis=0)
        e1_ref[...] = e1t.T
        pc2e_ref[...] = jnp.concatenate([pc2_ref[...], one], axis=0).T

    h = _TN1 // _NC
    i = pl.program_id(0)

    def _mm(k):
        sl = pl.ds(k * h, h)
        gsl = pl.ds(pl.multiple_of(i * _TN1 + k * h, h), h)
        d1n = d1_ref[:, sl] / n1_ref[:, sl]
        ip = jax.lax.dot_general(d1n, d2n_ref[...], (((0,), (0,)), ((), ())),
                                 preferred_element_type=jnp.float32)
        sqd = jnp.dot(e1_ref[gsl, :], e2_ref[...],
                      preferred_element_type=jnp.float32)
        return ip, sqd

    def _soft(ip, sqd):
        dist = jnp.reciprocal(jnp.maximum(sqd, _EPS)) * (ip * ip)
        m = jnp.max(dist, axis=1, keepdims=True)
        p = jnp.exp2((dist - m) * jnp.float32(_FACT * 1.4426950408889634))
        w = p / jnp.sum(p, axis=1, keepdims=True)
        return jnp.dot(w, pc2e_ref[...], preferred_element_type=jnp.float32)

    pend = {0: _mm(0), 1: _mm(1)}
    outs = []
    for k in range(_NC):
        if k + 2 < _NC:
            pend[k + 2] = _mm(k + 2)
        outs.append(_soft(*pend.pop(k)))
    out_ref[...] = jnp.concatenate(outs, axis=0)


def kernel(pc1, pc2, d1, d2):
    n = pc1.shape[1]
    f32 = jnp.float32
    n1 = jnp.maximum(jnp.linalg.norm(d1, axis=0, keepdims=True), 1e-12)
    n2 = jnp.maximum(jnp.linalg.norm(d2, axis=0, keepdims=True), 1e-12)
    s = d2.shape[0]

    out = pl.pallas_call(
        _body,
        grid=(n // _TN1,),
        in_specs=[
            pl.BlockSpec((3, n), lambda i: (0, 0)),
            pl.BlockSpec((3, n), lambda i: (0, 0)),
            pl.BlockSpec((s, _TN1), lambda i: (0, i)),
            pl.BlockSpec((1, _TN1), lambda i: (0, i)),
            pl.BlockSpec((s, n), lambda i: (0, 0)),
            pl.BlockSpec((1, n), lambda i: (0, 0)),
        ],
        out_specs=pl.BlockSpec((_TN1, 4), lambda i: (i, 0)),
        out_shape=jax.ShapeDtypeStruct((n, 4), f32),
        scratch_shapes=[
            pltpu.VMEM((s, n), f32),
            pltpu.VMEM((n, 9), f32),
            pltpu.VMEM((9, n), f32),
            pltpu.VMEM((n, 4), f32),
        ],
    )(pc1, pc2, d1, n1, d2, n2)

    pc_nearest = out[:, :3].T
    indexor = jnp.ones((n,), pc1.dtype)
    return (pc_nearest, indexor)
